# Initial kernel scaffold; baseline (speedup 1.0000x reference)
import jax
import jax.numpy as jnp
from jax.experimental import pallas as pl


def kernel(pos_u, pos_v, neg_v, u_table, v_table):
    raise NotImplementedError("write your pallas kernel here")


# speedup vs baseline: 1.5633x; 1.5633x over previous
import functools

import jax
import jax.numpy as jnp
from jax import lax
from jax.experimental import pallas as pl
from jax.experimental.pallas import tpu as pltpu
from jax.experimental.pallas import tpu_sc as plsc

_NC = 2
_NS = 16
_NW = _NC * _NS
_LANES = 16


def _sc_scores(pos_u, pos_v, neg_t, u_pair, v_pair, B, K, D, chunk):
    per_w = B // _NW
    nchunks = per_w // chunk
    ngroups = chunk // _LANES
    W = 2 * D
    mesh = plsc.VectorSubcoreMesh(core_axis_name="c", subcore_axis_name="s")

    @functools.partial(
        pl.kernel,
        out_type=jax.ShapeDtypeStruct(((1 + K) * B,), jnp.float32),
        mesh=mesh,
        compiler_params=pltpu.CompilerParams(needs_layout_passes=False),
        scratch_types=[
            pltpu.VMEM((chunk,), jnp.int32),
            pltpu.VMEM((chunk,), jnp.int32),
            pltpu.VMEM((K, chunk), jnp.int32),
            pltpu.VMEM((chunk,), jnp.int32),
            pltpu.VMEM((chunk,), jnp.int32),
            pltpu.VMEM((K, chunk), jnp.int32),
            pltpu.VMEM((chunk, 2 * D), jnp.float32),
            pltpu.VMEM((chunk, 2 * D), jnp.float32),
            pltpu.VMEM((K, chunk, 2 * D), jnp.float32),
            pltpu.VMEM((1 + K, chunk), jnp.float32),
            pltpu.SemaphoreType.DMA,
        ],
    )
    def scores_kernel(pos_u_hbm, pos_v_hbm, neg_t_hbm, u_hbm, v_hbm, out_hbm,
                      idx_u, idx_v, idx_n, idx_uh, idx_vh, idx_nh,
                      u_rows, v_rows, n_rows, scores, sem):
        wid = lax.axis_index("s") * _NC + lax.axis_index("c")
        base = wid * per_w
        zero16 = jnp.zeros((_LANES,), jnp.float32)

        def halve(src, dst):
            for g in range(ngroups):
                sl = pl.ds(g * _LANES, _LANES)
                dst[sl] = lax.shift_right_logical(src[sl], 1)

        def chunk_body(c, _):
            off = pl.multiple_of(base + c * chunk, chunk)
            pltpu.sync_copy(pos_u_hbm.at[pl.ds(off, chunk)], idx_u)
            pltpu.sync_copy(pos_v_hbm.at[pl.ds(off, chunk)], idx_v)
            for k in range(K):
                pltpu.sync_copy(neg_t_hbm.at[pl.ds(k * B + off, chunk)],
                                idx_n.at[k])
            halve(idx_u, idx_uh)
            halve(idx_v, idx_vh)
            for k in range(K):
                halve(idx_n.at[k], idx_nh.at[k])
            cps = [pltpu.async_copy(u_hbm.at[idx_uh], u_rows, sem),
                   pltpu.async_copy(v_hbm.at[idx_vh], v_rows, sem)]
            for k in range(K):
                cps.append(
                    pltpu.async_copy(v_hbm.at[idx_nh.at[k]], n_rows.at[k], sem))
            for cp in cps:
                cp.wait()

            def group_body(g, _):
                sl = pl.ds(g * _LANES, _LANES)
                row = g * _LANES + lax.iota(jnp.int32, _LANES)
                off_u = (idx_u[sl] & 1) * D
                off_v = (idx_v[sl] & 1) * D
                off_n = [(idx_n[k, sl] & 1) * D for k in range(K)]
                accs = [zero16] * (1 + K)
                for d in range(D):
                    u_col = plsc.load_gather(u_rows, [row, off_u + d])
                    accs[0] = accs[0] + u_col * plsc.load_gather(
                        v_rows, [row, off_v + d])
                    for k in range(K):
                        accs[1 + k] = accs[1 + k] + u_col * plsc.load_gather(
                            n_rows.at[k], [row, off_n[k] + d])
                for r in range(1 + K):
                    scores[r, sl] = accs[r]
                return 0

            lax.fori_loop(0, ngroups, group_body, 0)
            for r in range(1 + K):
                pltpu.sync_copy(scores.at[r], out_hbm.at[pl.ds(r * B + off, chunk)])
            return 0

        lax.fori_loop(0, nchunks, chunk_body, 0)

    return scores_kernel(pos_u, pos_v, neg_t, u_pair, v_pair)


def _loss_body(s_ref, o_ref):
    s = s_ref[...]
    pos = s[0:1, :]
    neg = s[1:, :]

    def logsig(x):
        return jnp.minimum(x, 0.0) - jnp.log1p(jnp.exp(-jnp.abs(x)))

    total = jnp.sum(logsig(pos)) + jnp.sum(logsig(-neg))
    o_ref[...] = (-total).reshape(1, 1)


def kernel(pos_u, pos_v, neg_v, u_table, v_table):
    B = pos_u.shape[0]
    K = neg_v.shape[1]
    V, D = u_table.shape
    pos_u = pos_u.astype(jnp.int32)
    pos_v = pos_v.astype(jnp.int32)
    neg_t = neg_v.astype(jnp.int32).T.reshape(-1)

    u_pair = u_table.reshape(V // 2, 2 * D)
    v_pair = v_table.reshape(V // 2, 2 * D)

    scores = _sc_scores(pos_u, pos_v, neg_t, u_pair, v_pair, B, K, D,
                        chunk=128).reshape(1 + K, B)

    loss = pl.pallas_call(
        _loss_body,
        out_shape=jax.ShapeDtypeStruct((1, 1), jnp.float32),
    )(scores)
    return loss[0, 0]


# speedup vs baseline: 1.5737x; 1.0066x over previous
import functools

import jax
import jax.numpy as jnp
from jax import lax
from jax.experimental import pallas as pl
from jax.experimental.pallas import tpu as pltpu
from jax.experimental.pallas import tpu_sc as plsc

_NC = 2
_NS = 16
_NW = _NC * _NS
_LANES = 16


def _sc_scores(pos_u, pos_v, neg_t, u_pair, v_pair, B, K, D, chunk):
    per_w = B // _NW
    nchunks = per_w // chunk
    ngroups = chunk // _LANES
    W = 2 * D
    mesh = plsc.VectorSubcoreMesh(core_axis_name="c", subcore_axis_name="s")

    @functools.partial(
        pl.kernel,
        out_type=jax.ShapeDtypeStruct(((1 + K) * B,), jnp.float32),
        mesh=mesh,
        compiler_params=pltpu.CompilerParams(needs_layout_passes=False),
        scratch_types=[
            pltpu.VMEM((chunk,), jnp.int32),
            pltpu.VMEM((chunk,), jnp.int32),
            pltpu.VMEM((K, chunk), jnp.int32),
            pltpu.VMEM((chunk,), jnp.int32),
            pltpu.VMEM((chunk,), jnp.int32),
            pltpu.VMEM((K, chunk), jnp.int32),
            pltpu.VMEM((chunk, 2 * D), jnp.float32),
            pltpu.VMEM((chunk, 2 * D), jnp.float32),
            pltpu.VMEM((K, chunk, 2 * D), jnp.float32),
            pltpu.VMEM((1 + K, chunk), jnp.float32),
            pltpu.SemaphoreType.DMA,
        ],
    )
    def scores_kernel(pos_u_hbm, pos_v_hbm, neg_t_hbm, u_hbm, v_hbm, out_hbm,
                      idx_u, idx_v, idx_n, idx_uh, idx_vh, idx_nh,
                      u_rows, v_rows, n_rows, scores, sem):
        wid = lax.axis_index("s") * _NC + lax.axis_index("c")
        base = wid * per_w
        zero16 = jnp.zeros((_LANES,), jnp.float32)

        def halve(src, dst):
            for g in range(ngroups):
                sl = pl.ds(g * _LANES, _LANES)
                dst[sl] = lax.shift_right_logical(src[sl], 1)

        def chunk_body(c, _):
            off = pl.multiple_of(base + c * chunk, chunk)
            pltpu.sync_copy(pos_u_hbm.at[pl.ds(off, chunk)], idx_u)
            pltpu.sync_copy(pos_v_hbm.at[pl.ds(off, chunk)], idx_v)
            pltpu.sync_copy(neg_t_hbm.at[:, pl.ds(off, chunk)], idx_n)
            halve(idx_u, idx_uh)
            halve(idx_v, idx_vh)
            for k in range(K):
                halve(idx_n.at[k], idx_nh.at[k])
            cps = [pltpu.async_copy(u_hbm.at[idx_uh], u_rows, sem),
                   pltpu.async_copy(v_hbm.at[idx_vh], v_rows, sem)]
            for k in range(K):
                cps.append(
                    pltpu.async_copy(v_hbm.at[idx_nh.at[k]], n_rows.at[k], sem))
            for cp in cps:
                cp.wait()

            def group_body(g, _):
                sl = pl.ds(g * _LANES, _LANES)
                row = g * _LANES + lax.iota(jnp.int32, _LANES)
                off_u = (idx_u[sl] & 1) * D
                off_v = (idx_v[sl] & 1) * D
                off_n = [(idx_n[k, sl] & 1) * D for k in range(K)]
                accs = [zero16] * (1 + K)
                for d in range(D):
                    u_col = plsc.load_gather(u_rows, [row, off_u + d])
                    accs[0] = accs[0] + u_col * plsc.load_gather(
                        v_rows, [row, off_v + d])
                    for k in range(K):
                        accs[1 + k] = accs[1 + k] + u_col * plsc.load_gather(
                            n_rows.at[k], [row, off_n[k] + d])
                for r in range(1 + K):
                    scores[r, sl] = accs[r]
                return 0

            lax.fori_loop(0, ngroups, group_body, 0)
            for r in range(1 + K):
                pltpu.sync_copy(scores.at[r], out_hbm.at[pl.ds(r * B + off, chunk)])
            return 0

        lax.fori_loop(0, nchunks, chunk_body, 0)

    return scores_kernel(pos_u, pos_v, neg_t, u_pair, v_pair)


def _loss_body(s_ref, o_ref):
    s = s_ref[...]
    pos = s[0:1, :]
    neg = s[1:, :]

    def logsig(x):
        return jnp.minimum(x, 0.0) - jnp.log1p(jnp.exp(-jnp.abs(x)))

    total = jnp.sum(logsig(pos)) + jnp.sum(logsig(-neg))
    o_ref[...] = (-total).reshape(1, 1)


def kernel(pos_u, pos_v, neg_v, u_table, v_table):
    B = pos_u.shape[0]
    K = neg_v.shape[1]
    V, D = u_table.shape
    pos_u = pos_u.astype(jnp.int32)
    pos_v = pos_v.astype(jnp.int32)
    neg_t = neg_v.astype(jnp.int32).T

    u_pair = u_table.reshape(V // 2, 2 * D)
    v_pair = v_table.reshape(V // 2, 2 * D)

    scores = _sc_scores(pos_u, pos_v, neg_t, u_pair, v_pair, B, K, D,
                        chunk=128).reshape(1 + K, B)

    loss = pl.pallas_call(
        _loss_body,
        out_shape=jax.ShapeDtypeStruct((1, 1), jnp.float32),
    )(scores)
    return loss[0, 0]
